# baseline (speedup 1.0000x reference)
import functools

import jax
import jax.numpy as jnp
from jax import lax
from jax.experimental import pallas as pl
from jax.experimental.pallas import tpu as pltpu
from jax.experimental.pallas import tpu_sc as plsc

N = 10000
E = 320000
D = 128

NC = 2
NS = 16
NW = NC * NS
CHUNK = 128
NCHUNK = 80
EPW = NCHUNK * CHUNK
E_PAD = NW * EPW
NP = 10240
RPT = NP // NS
GBUF = 2
IBUF = 4

_mesh = plsc.VectorSubcoreMesh(
    core_axis_name="c", subcore_axis_name="s", num_cores=NC, num_subcores=NS
)


def _agg_body(with_cnt, *refs):
    if with_cnt:
        (x_hbm, src_hbm, dst_hbm, zacc_hbm,
         acc_out, cnt_out,
         acc_sh, cnt_sh, sib, dib, rows, ones_v, cv,
         sem_si, sem_di, sem_g, sem_s, sem_c) = refs
    else:
        (x_hbm, src_hbm, dst_hbm, zacc_hbm,
         acc_out,
         acc_sh, sib, dib, rows, sem_si, sem_di, sem_g, sem_s) = refs

    cid = lax.axis_index("c")
    sid = lax.axis_index("s")
    wid = cid * NS + sid

    r0 = pl.multiple_of(sid * RPT, 8)
    pltpu.sync_copy(zacc_hbm.at[pl.ds(r0, RPT)], acc_sh.at[pl.ds(r0, RPT)])
    if with_cnt:
        z16 = jnp.zeros((16,), jnp.float32)
        o16 = jnp.ones((16,), jnp.float32)

        def zrow(r, c):
            cv[pl.ds(r * 16, 16)] = z16
            return c

        lax.fori_loop(0, RPT // 16, zrow, 0)

        def orow(r, c):
            ones_v[pl.ds(r * 16, 16)] = o16
            return c

        lax.fori_loop(0, CHUNK // 16, orow, 0)
        pltpu.sync_copy(cv, cnt_sh.at[pl.ds(r0, RPT)])
    plsc.subcore_barrier()

    base0 = wid * EPW

    def i_descs(i, j):
        base = pl.multiple_of(base0 + i * CHUNK, 8)
        return (
            pltpu.make_async_copy(
                src_hbm.at[pl.ds(base, CHUNK)], sib.at[j], sem_si[j]),
            pltpu.make_async_copy(
                dst_hbm.at[pl.ds(base, CHUNK)], dib.at[j], sem_di[j]),
        )

    def i_start(i, j):
        s, d = i_descs(i, j)
        s.start()
        d.start()

    def i_wait(i, j):
        s, d = i_descs(i, j)
        s.wait()
        d.wait()

    def g_desc(j, b):
        return pltpu.make_async_copy(x_hbm.at[sib.at[j]], rows[b], sem_g[b])

    def s_desc(j, b):
        return pltpu.make_async_copy(
            rows[b], acc_sh.at[dib.at[j]], sem_s[b])

    def c_desc(j, b):
        return pltpu.make_async_copy(ones_v, cnt_sh.at[dib.at[j]], sem_c[b])

    def s_start(j, b):
        s_desc(j, b).start(add=True)
        if with_cnt:
            c_desc(j, b).start(add=True)

    def s_wait(j, b):
        s_desc(j, b).wait()
        if with_cnt:
            c_desc(j, b).wait()

    for i in range(IBUF):
        i_start(i, i)
    i_wait(0, 0)
    g_desc(0, 0).start()
    g_desc(0, 0).wait()
    s_start(0, 0)
    i_wait(1, 1)
    g_desc(1, 1).start()

    @pl.loop(1, NCHUNK - IBUF + 1, step=IBUF)
    def ring(i0):
        for u in range(IBUF):
            i = i0 + u
            ji = (1 + u) % IBUF
            b = (1 + u) % GBUF
            jp = u % IBUF
            bp = u % GBUF
            jn = (2 + u) % IBUF
            g_desc(ji, b).wait()
            s_start(ji, b)
            s_wait(jp, bp)
            i_start(i + IBUF - 1, jp)
            i_wait(i + 1, jn)
            g_desc(jn, bp).start()

    for i in range(NCHUNK - IBUF + 1, NCHUNK):
        g_desc(i % IBUF, i % GBUF).wait()
        s_start(i % IBUF, i % GBUF)
        s_wait((i - 1) % IBUF, (i - 1) % GBUF)
        if i + 1 < NCHUNK:
            i_wait(i + 1, (i + 1) % IBUF)
            g_desc((i + 1) % IBUF, (i + 1) % GBUF).start()
    s_wait((NCHUNK - 1) % IBUF, (NCHUNK - 1) % GBUF)
    plsc.subcore_barrier()

    pltpu.sync_copy(acc_sh.at[pl.ds(r0, RPT)], acc_out.at[cid, pl.ds(r0, RPT)])
    if with_cnt:
        pltpu.sync_copy(cnt_sh.at[pl.ds(r0, RPT)], cv)
        pltpu.sync_copy(cv, cnt_out.at[pl.ds(cid * NP + r0, RPT)])


_agg_with_cnt = pl.kernel(
    functools.partial(_agg_body, True),
    out_type=(
        jax.ShapeDtypeStruct((NC, NP, D), jnp.float32),
        jax.ShapeDtypeStruct((NC * NP,), jnp.float32),
    ),
    mesh=_mesh,
    scratch_types=[
        pltpu.VMEM_SHARED((NP, D), jnp.float32),
        pltpu.VMEM_SHARED((NP,), jnp.float32),
        pltpu.VMEM((IBUF, CHUNK), jnp.int32),
        pltpu.VMEM((IBUF, CHUNK), jnp.int32),
        [pltpu.VMEM((CHUNK, D), jnp.float32) for _ in range(GBUF)],
        pltpu.VMEM((CHUNK,), jnp.float32),
        pltpu.VMEM((RPT,), jnp.float32),
        [pltpu.SemaphoreType.DMA for _ in range(IBUF)],
        [pltpu.SemaphoreType.DMA for _ in range(IBUF)],
        [pltpu.SemaphoreType.DMA for _ in range(GBUF)],
        [pltpu.SemaphoreType.DMA for _ in range(GBUF)],
        [pltpu.SemaphoreType.DMA for _ in range(GBUF)],
    ],
    name="sage_agg_cnt",
)

_agg_no_cnt = pl.kernel(
    functools.partial(_agg_body, False),
    out_type=jax.ShapeDtypeStruct((NC, NP, D), jnp.float32),
    mesh=_mesh,
    scratch_types=[
        pltpu.VMEM_SHARED((NP, D), jnp.float32),
        pltpu.VMEM((IBUF, CHUNK), jnp.int32),
        pltpu.VMEM((IBUF, CHUNK), jnp.int32),
        [pltpu.VMEM((CHUNK, D), jnp.float32) for _ in range(GBUF)],
        [pltpu.SemaphoreType.DMA for _ in range(IBUF)],
        [pltpu.SemaphoreType.DMA for _ in range(IBUF)],
        [pltpu.SemaphoreType.DMA for _ in range(GBUF)],
        [pltpu.SemaphoreType.DMA for _ in range(GBUF)],
    ],
    name="sage_agg",
)

BN = 400


def _combine_body(p_ref, c_ref, x_ref, wl_ref, wr_ref, b_ref, o_ref):
    cnt = jnp.maximum(c_ref[0] + c_ref[1], 1.0)
    agg = (p_ref[0] + p_ref[1]) / cnt
    acc = jax.lax.dot_general(
        agg, wl_ref[...], (((1,), (0,)), ((), ())),
        preferred_element_type=jnp.float32,
        precision=jax.lax.Precision.HIGHEST)
    acc = acc + jax.lax.dot_general(
        x_ref[...], wr_ref[...], (((1,), (0,)), ((), ())),
        preferred_element_type=jnp.float32,
        precision=jax.lax.Precision.HIGHEST)
    o_ref[...] = jnp.maximum(acc + b_ref[...], 0.0)


def _combine(p, c, x, W_l, W_r, b):
    return pl.pallas_call(
        _combine_body,
        grid=(N // BN,),
        in_specs=[
            pl.BlockSpec((NC, BN, D), lambda i: (0, i, 0)),
            pl.BlockSpec((NC, BN, 1), lambda i: (0, i, 0)),
            pl.BlockSpec((BN, D), lambda i: (i, 0)),
            pl.BlockSpec((D, D), lambda i: (0, 0)),
            pl.BlockSpec((D, D), lambda i: (0, 0)),
            pl.BlockSpec((1, D), lambda i: (0, 0)),
        ],
        out_specs=pl.BlockSpec((BN, D), lambda i: (i, 0)),
        out_shape=jax.ShapeDtypeStruct((N, D), jnp.float32),
    )(p, c, x, W_l, W_r, b)


@jax.jit
def kernel(x, edge_index, W1_l, W1_r, b1, W2_l, W2_r, b2):
    src = edge_index[0]
    dst = edge_index[1]
    npad = E_PAD - E
    src_p = jnp.concatenate([src, jnp.zeros((npad,), jnp.int32)])
    dst_p = jnp.concatenate(
        [dst, N + jnp.arange(npad, dtype=jnp.int32) % (NP - N)])
    zacc = jnp.zeros((NP, D), jnp.float32)
    b1r = b1.reshape(1, D)
    b2r = b2.reshape(1, D)

    p1, cnt_flat = _agg_with_cnt(x, src_p, dst_p, zacc)
    cnt = cnt_flat.reshape(NC, NP, 1)
    h = _combine(p1, cnt, x, W1_l, W1_r, b1r)
    p2 = _agg_no_cnt(h, src_p, dst_p, zacc)
    return _combine(p2, cnt, h, W2_l, W2_r, b2r)


# speedup vs baseline: 3.0939x; 3.0939x over previous
import functools

import jax
import jax.numpy as jnp
from jax import lax
from jax.experimental import pallas as pl
from jax.experimental.pallas import tpu as pltpu
from jax.experimental.pallas import tpu_sc as plsc

N = 10000
E = 320000
D = 128

NC = 2
NS = 16
NW = NC * NS
CHUNK = 128
NCHUNK = 80
EPW = NCHUNK * CHUNK
E_PAD = NW * EPW
NP = 10240
RPT = NP // NS
GBUF = 2
IBUF = 4

_mesh = plsc.VectorSubcoreMesh(
    core_axis_name="c", subcore_axis_name="s", num_cores=NC, num_subcores=NS
)


def _agg_body(with_cnt, *refs):
    if with_cnt:
        (x_hbm, src_hbm, dst_hbm, zacc_hbm,
         acc_out, cnt_out,
         acc_sh, cnt_sh, sib, dib, rows, ones_v, cv,
         sem_si, sem_di, sem_g, sem_s, sem_c) = refs
    else:
        (x_hbm, src_hbm, dst_hbm, zacc_hbm,
         acc_out,
         acc_sh, sib, dib, rows, sem_si, sem_di, sem_g, sem_s) = refs

    cid = lax.axis_index("c")
    sid = lax.axis_index("s")
    wid = cid * NS + sid

    r0 = pl.multiple_of(sid * RPT, 8)
    pltpu.sync_copy(zacc_hbm.at[pl.ds(r0, RPT)], acc_sh.at[pl.ds(r0, RPT)])
    if with_cnt:
        z16 = jnp.zeros((16,), jnp.float32)
        o16 = jnp.ones((16,), jnp.float32)

        def zrow(r, c):
            cv[pl.ds(r * 16, 16)] = z16
            return c

        lax.fori_loop(0, RPT // 16, zrow, 0)

        def orow(r, c):
            ones_v[pl.ds(r * 16, 16)] = o16
            return c

        lax.fori_loop(0, CHUNK // 16, orow, 0)
        pltpu.sync_copy(cv, cnt_sh.at[pl.ds(r0, RPT)])
    plsc.subcore_barrier()

    base0 = wid * EPW

    def i_descs(i, j):
        base = pl.multiple_of(base0 + i * CHUNK, 8)
        return (
            pltpu.make_async_copy(
                src_hbm.at[pl.ds(base, CHUNK)], sib.at[j], sem_si[j]),
            pltpu.make_async_copy(
                dst_hbm.at[pl.ds(base, CHUNK)], dib.at[j], sem_di[j]),
        )

    def i_start(i, j):
        s, d = i_descs(i, j)
        s.start()
        d.start()

    def i_wait(i, j):
        s, d = i_descs(i, j)
        s.wait()
        d.wait()

    def g_desc(j, b):
        return pltpu.make_async_copy(x_hbm.at[sib.at[j]], rows[b], sem_g[b])

    def s_desc(j, b):
        return pltpu.make_async_copy(
            rows[b], acc_sh.at[dib.at[j]], sem_s[b])

    def c_desc(j, b):
        return pltpu.make_async_copy(ones_v, cnt_sh.at[dib.at[j]], sem_c[b])

    def s_start(j, b):
        s_desc(j, b).start(add=True)
        if with_cnt:
            c_desc(j, b).start(add=True)

    def s_wait(j, b):
        s_desc(j, b).wait()
        if with_cnt:
            c_desc(j, b).wait()

    for i in range(IBUF):
        i_start(i, i)
    i_wait(0, 0)
    g_desc(0, 0).start()
    g_desc(0, 0).wait()
    s_start(0, 0)
    i_wait(1, 1)
    g_desc(1, 1).start()

    @pl.loop(1, NCHUNK - IBUF + 1, step=IBUF)
    def ring(i0):
        for u in range(IBUF):
            i = i0 + u
            ji = (1 + u) % IBUF
            b = (1 + u) % GBUF
            jp = u % IBUF
            bp = u % GBUF
            jn = (2 + u) % IBUF
            g_desc(ji, b).wait()
            s_start(ji, b)
            s_wait(jp, bp)
            i_start(i + IBUF - 1, jp)
            i_wait(i + 1, jn)
            g_desc(jn, bp).start()

    for i in range(NCHUNK - IBUF + 1, NCHUNK):
        g_desc(i % IBUF, i % GBUF).wait()
        s_start(i % IBUF, i % GBUF)
        s_wait((i - 1) % IBUF, (i - 1) % GBUF)
        if i + 1 < NCHUNK:
            i_wait(i + 1, (i + 1) % IBUF)
            g_desc((i + 1) % IBUF, (i + 1) % GBUF).start()
    s_wait((NCHUNK - 1) % IBUF, (NCHUNK - 1) % GBUF)
    plsc.subcore_barrier()

    pltpu.sync_copy(acc_sh.at[pl.ds(r0, RPT)], acc_out.at[cid, pl.ds(r0, RPT)])
    if with_cnt:
        pltpu.sync_copy(cnt_sh.at[pl.ds(r0, RPT)], cv)
        pltpu.sync_copy(cv, cnt_out.at[pl.ds(cid * NP + r0, RPT)])


_agg_with_cnt = pl.kernel(
    functools.partial(_agg_body, True),
    out_type=(
        jax.ShapeDtypeStruct((NC, NP, D), jnp.float32),
        jax.ShapeDtypeStruct((NC * NP,), jnp.float32),
    ),
    mesh=_mesh,
    scratch_types=[
        pltpu.VMEM_SHARED((NP, D), jnp.float32),
        pltpu.VMEM_SHARED((NP,), jnp.float32),
        pltpu.VMEM((IBUF, CHUNK), jnp.int32),
        pltpu.VMEM((IBUF, CHUNK), jnp.int32),
        [pltpu.VMEM((CHUNK, D), jnp.float32) for _ in range(GBUF)],
        pltpu.VMEM((CHUNK,), jnp.float32),
        pltpu.VMEM((RPT,), jnp.float32),
        [pltpu.SemaphoreType.DMA for _ in range(IBUF)],
        [pltpu.SemaphoreType.DMA for _ in range(IBUF)],
        [pltpu.SemaphoreType.DMA for _ in range(GBUF)],
        [pltpu.SemaphoreType.DMA for _ in range(GBUF)],
        [pltpu.SemaphoreType.DMA for _ in range(GBUF)],
    ],
    name="sage_agg_cnt",
)

_agg_no_cnt = pl.kernel(
    functools.partial(_agg_body, False),
    out_type=jax.ShapeDtypeStruct((NC, NP, D), jnp.float32),
    mesh=_mesh,
    scratch_types=[
        pltpu.VMEM_SHARED((NP, D), jnp.float32),
        pltpu.VMEM((IBUF, CHUNK), jnp.int32),
        pltpu.VMEM((IBUF, CHUNK), jnp.int32),
        [pltpu.VMEM((CHUNK, D), jnp.float32) for _ in range(GBUF)],
        [pltpu.SemaphoreType.DMA for _ in range(IBUF)],
        [pltpu.SemaphoreType.DMA for _ in range(IBUF)],
        [pltpu.SemaphoreType.DMA for _ in range(GBUF)],
        [pltpu.SemaphoreType.DMA for _ in range(GBUF)],
    ],
    name="sage_agg",
)

BN = 400


def _combine_body(p_ref, c_ref, x_ref, wl_ref, wr_ref, b_ref, o_ref):
    cnt = jnp.maximum(c_ref[0] + c_ref[1], 1.0)
    agg = (p_ref[0] + p_ref[1]) / cnt
    acc = jax.lax.dot_general(
        agg, wl_ref[...], (((1,), (0,)), ((), ())),
        preferred_element_type=jnp.float32,
        precision=jax.lax.Precision.HIGHEST)
    acc = acc + jax.lax.dot_general(
        x_ref[...], wr_ref[...], (((1,), (0,)), ((), ())),
        preferred_element_type=jnp.float32,
        precision=jax.lax.Precision.HIGHEST)
    o_ref[...] = jnp.maximum(acc + b_ref[...], 0.0)


def _combine(p, c, x, W_l, W_r, b):
    return pl.pallas_call(
        _combine_body,
        grid=(N // BN,),
        in_specs=[
            pl.BlockSpec((NC, BN, D), lambda i: (0, i, 0)),
            pl.BlockSpec((NC, BN, 1), lambda i: (0, i, 0)),
            pl.BlockSpec((BN, D), lambda i: (i, 0)),
            pl.BlockSpec((D, D), lambda i: (0, 0)),
            pl.BlockSpec((D, D), lambda i: (0, 0)),
            pl.BlockSpec((1, D), lambda i: (0, 0)),
        ],
        out_specs=pl.BlockSpec((BN, D), lambda i: (i, 0)),
        out_shape=jax.ShapeDtypeStruct((N, D), jnp.float32),
    )(p, c, x, W_l, W_r, b)


@jax.jit
def kernel(x, edge_index, W1_l, W1_r, b1, W2_l, W2_r, b2):
    src = edge_index[0]
    dst = edge_index[1]
    npad = E_PAD - E
    src_p = jnp.concatenate([src, jnp.arange(npad, dtype=jnp.int32) % N])
    dst_p = jnp.concatenate(
        [dst, N + jnp.arange(npad, dtype=jnp.int32) % (NP - N)])
    zacc = jnp.zeros((NP, D), jnp.float32)
    b1r = b1.reshape(1, D)
    b2r = b2.reshape(1, D)

    p1, cnt_flat = _agg_with_cnt(x, src_p, dst_p, zacc)
    cnt = cnt_flat.reshape(NC, NP, 1)
    h = _combine(p1, cnt, x, W1_l, W1_r, b1r)
    p2 = _agg_no_cnt(h, src_p, dst_p, zacc)
    return _combine(p2, cnt, h, W2_l, W2_r, b2r)


# speedup vs baseline: 3.1963x; 1.0331x over previous
import functools

import jax
import jax.numpy as jnp
from jax import lax
from jax.experimental import pallas as pl
from jax.experimental.pallas import tpu as pltpu
from jax.experimental.pallas import tpu_sc as plsc

N = 10000
E = 320000
D = 128

NC = 2
NS = 16
NW = NC * NS
CHUNK = 128
EPW = E // NW
NF = EPW // CHUNK
TAIL = EPW - NF * CHUNK
NP = 10240
RPT = NP // NS
GBUF = 2
IBUF = 4

_mesh = plsc.VectorSubcoreMesh(
    core_axis_name="c", subcore_axis_name="s", num_cores=NC, num_subcores=NS
)


def _agg_body(with_cnt, *refs):
    if with_cnt:
        (x_hbm, src_hbm, dst_hbm, zacc_hbm,
         acc_out, cnt_out,
         acc_sh, cnt_sh, sib, dib, rows, sit, dit, rows_t, ones_v, cv,
         sem_si, sem_di, sem_g, sem_s, sem_c, sem_t) = refs
    else:
        (x_hbm, src_hbm, dst_hbm, zacc_hbm,
         acc_out,
         acc_sh, sib, dib, rows, sit, dit, rows_t,
         sem_si, sem_di, sem_g, sem_s, sem_t) = refs

    cid = lax.axis_index("c")
    sid = lax.axis_index("s")
    wid = cid * NS + sid

    r0 = pl.multiple_of(sid * RPT, 8)
    pltpu.sync_copy(zacc_hbm.at[pl.ds(r0, RPT)], acc_sh.at[pl.ds(r0, RPT)])
    if with_cnt:
        z16 = jnp.zeros((16,), jnp.float32)
        o16 = jnp.ones((16,), jnp.float32)

        def zrow(r, c):
            cv[pl.ds(r * 16, 16)] = z16
            return c

        lax.fori_loop(0, RPT // 16, zrow, 0)

        def orow(r, c):
            ones_v[pl.ds(r * 16, 16)] = o16
            return c

        lax.fori_loop(0, CHUNK // 16, orow, 0)
        pltpu.sync_copy(cv, cnt_sh.at[pl.ds(r0, RPT)])
    plsc.subcore_barrier()

    base0 = wid * EPW

    def i_descs(i, j):
        base = pl.multiple_of(base0 + i * CHUNK, 8)
        return (
            pltpu.make_async_copy(
                src_hbm.at[pl.ds(base, CHUNK)], sib.at[j], sem_si[j]),
            pltpu.make_async_copy(
                dst_hbm.at[pl.ds(base, CHUNK)], dib.at[j], sem_di[j]),
        )

    def i_start(i, j):
        s, d = i_descs(i, j)
        s.start()
        d.start()

    def i_wait(i, j):
        s, d = i_descs(i, j)
        s.wait()
        d.wait()

    def g_desc(j, b):
        return pltpu.make_async_copy(x_hbm.at[sib.at[j]], rows[b], sem_g[b])

    def s_desc(j, b):
        return pltpu.make_async_copy(
            rows[b], acc_sh.at[dib.at[j]], sem_s[b])

    def c_desc(j, b):
        return pltpu.make_async_copy(ones_v, cnt_sh.at[dib.at[j]], sem_c[b])

    def s_start(j, b):
        s_desc(j, b).start(add=True)
        if with_cnt:
            c_desc(j, b).start(add=True)

    def s_wait(j, b):
        s_desc(j, b).wait()
        if with_cnt:
            c_desc(j, b).wait()

    tbase = pl.multiple_of(base0 + NF * CHUNK, 8)
    t_si = pltpu.make_async_copy(
        src_hbm.at[pl.ds(tbase, TAIL)], sit, sem_t[0])
    t_di = pltpu.make_async_copy(
        dst_hbm.at[pl.ds(tbase, TAIL)], dit, sem_t[1])
    t_g = pltpu.make_async_copy(x_hbm.at[sit], rows_t, sem_t[2])
    t_s = pltpu.make_async_copy(rows_t, acc_sh.at[dit], sem_t[3])
    if with_cnt:
        t_c = pltpu.make_async_copy(
            ones_v.at[pl.ds(0, TAIL)], cnt_sh.at[dit], sem_t[4])

    t_si.start()
    t_di.start()
    for i in range(IBUF):
        i_start(i, i)
    t_si.wait()
    t_di.wait()
    t_g.start()
    i_wait(0, 0)
    g_desc(0, 0).start()
    i_wait(1, 1)
    g_desc(0, 0).wait()
    s_start(0, 0)
    g_desc(1, 1).start()
    for i in (1, 2):
        g_desc(i % IBUF, i % GBUF).wait()
        s_start(i % IBUF, i % GBUF)
        s_wait((i - 1) % IBUF, (i - 1) % GBUF)
        i_start(i + 3, (i + 3) % IBUF)
        i_wait(i + 1, (i + 1) % IBUF)
        g_desc((i + 1) % IBUF, (i + 1) % GBUF).start()

    @pl.loop(3, NF - IBUF + 1, step=IBUF)
    def ring(i0):
        for u in range(IBUF):
            i = i0 + u
            ji = (3 + u) % IBUF
            b = (3 + u) % GBUF
            jp = (2 + u) % IBUF
            bp = (2 + u) % GBUF
            jn = u % IBUF
            bn = u % GBUF
            g_desc(ji, b).wait()
            s_start(ji, b)
            s_wait(jp, bp)
            i_start(i + IBUF - 1, jp)
            i_wait(i + 1, jn)
            g_desc(jn, bn).start()

    for i in range(NF - IBUF + 1, NF):
        g_desc(i % IBUF, i % GBUF).wait()
        s_start(i % IBUF, i % GBUF)
        s_wait((i - 1) % IBUF, (i - 1) % GBUF)
        if i + 1 < NF:
            i_wait(i + 1, (i + 1) % IBUF)
            g_desc((i + 1) % IBUF, (i + 1) % GBUF).start()
    t_g.wait()
    t_s.start(add=True)
    if with_cnt:
        t_c.start(add=True)
    s_wait((NF - 1) % IBUF, (NF - 1) % GBUF)
    t_s.wait()
    if with_cnt:
        t_c.wait()
    plsc.subcore_barrier()

    pltpu.sync_copy(acc_sh.at[pl.ds(r0, RPT)], acc_out.at[cid, pl.ds(r0, RPT)])
    if with_cnt:
        pltpu.sync_copy(cnt_sh.at[pl.ds(r0, RPT)], cv)
        pltpu.sync_copy(cv, cnt_out.at[pl.ds(cid * NP + r0, RPT)])


_agg_with_cnt = pl.kernel(
    functools.partial(_agg_body, True),
    out_type=(
        jax.ShapeDtypeStruct((NC, NP, D), jnp.float32),
        jax.ShapeDtypeStruct((NC * NP,), jnp.float32),
    ),
    mesh=_mesh,
    scratch_types=[
        pltpu.VMEM_SHARED((NP, D), jnp.float32),
        pltpu.VMEM_SHARED((NP,), jnp.float32),
        pltpu.VMEM((IBUF, CHUNK), jnp.int32),
        pltpu.VMEM((IBUF, CHUNK), jnp.int32),
        [pltpu.VMEM((CHUNK, D), jnp.float32) for _ in range(GBUF)],
        pltpu.VMEM((TAIL,), jnp.int32),
        pltpu.VMEM((TAIL,), jnp.int32),
        pltpu.VMEM((TAIL, D), jnp.float32),
        pltpu.VMEM((CHUNK,), jnp.float32),
        pltpu.VMEM((RPT,), jnp.float32),
        [pltpu.SemaphoreType.DMA for _ in range(IBUF)],
        [pltpu.SemaphoreType.DMA for _ in range(IBUF)],
        [pltpu.SemaphoreType.DMA for _ in range(GBUF)],
        [pltpu.SemaphoreType.DMA for _ in range(GBUF)],
        [pltpu.SemaphoreType.DMA for _ in range(GBUF)],
        [pltpu.SemaphoreType.DMA for _ in range(5)],
    ],
    name="sage_agg_cnt",
)

_agg_no_cnt = pl.kernel(
    functools.partial(_agg_body, False),
    out_type=jax.ShapeDtypeStruct((NC, NP, D), jnp.float32),
    mesh=_mesh,
    scratch_types=[
        pltpu.VMEM_SHARED((NP, D), jnp.float32),
        pltpu.VMEM((IBUF, CHUNK), jnp.int32),
        pltpu.VMEM((IBUF, CHUNK), jnp.int32),
        [pltpu.VMEM((CHUNK, D), jnp.float32) for _ in range(GBUF)],
        pltpu.VMEM((TAIL,), jnp.int32),
        pltpu.VMEM((TAIL,), jnp.int32),
        pltpu.VMEM((TAIL, D), jnp.float32),
        [pltpu.SemaphoreType.DMA for _ in range(IBUF)],
        [pltpu.SemaphoreType.DMA for _ in range(IBUF)],
        [pltpu.SemaphoreType.DMA for _ in range(GBUF)],
        [pltpu.SemaphoreType.DMA for _ in range(GBUF)],
        [pltpu.SemaphoreType.DMA for _ in range(4)],
    ],
    name="sage_agg",
)

BN = 400


def _combine_body(p_ref, c_ref, x_ref, wl_ref, wr_ref, b_ref, o_ref):
    cnt = jnp.maximum(c_ref[0] + c_ref[1], 1.0)
    agg = (p_ref[0] + p_ref[1]) / cnt
    acc = jax.lax.dot_general(
        agg, wl_ref[...], (((1,), (0,)), ((), ())),
        preferred_element_type=jnp.float32)
    acc = acc + jax.lax.dot_general(
        x_ref[...], wr_ref[...], (((1,), (0,)), ((), ())),
        preferred_element_type=jnp.float32)
    o_ref[...] = jnp.maximum(acc + b_ref[...], 0.0)


def _combine(p, c, x, W_l, W_r, b):
    return pl.pallas_call(
        _combine_body,
        grid=(N // BN,),
        in_specs=[
            pl.BlockSpec((NC, BN, D), lambda i: (0, i, 0)),
            pl.BlockSpec((NC, BN, 1), lambda i: (0, i, 0)),
            pl.BlockSpec((BN, D), lambda i: (i, 0)),
            pl.BlockSpec((D, D), lambda i: (0, 0)),
            pl.BlockSpec((D, D), lambda i: (0, 0)),
            pl.BlockSpec((1, D), lambda i: (0, 0)),
        ],
        out_specs=pl.BlockSpec((BN, D), lambda i: (i, 0)),
        out_shape=jax.ShapeDtypeStruct((N, D), jnp.float32),
    )(p, c, x, W_l, W_r, b)


@jax.jit
def kernel(x, edge_index, W1_l, W1_r, b1, W2_l, W2_r, b2):
    src = edge_index[0]
    dst = edge_index[1]
    zacc = jnp.zeros((NP, D), jnp.float32)
    b1r = b1.reshape(1, D)
    b2r = b2.reshape(1, D)

    p1, cnt_flat = _agg_with_cnt(x, src, dst, zacc)
    cnt = cnt_flat.reshape(NC, NP, 1)
    h = _combine(p1, cnt, x, W1_l, W1_r, b1r)
    p2 = _agg_no_cnt(h, src, dst, zacc)
    return _combine(p2, cnt, h, W2_l, W2_r, b2r)


# speedup vs baseline: 3.2764x; 1.0251x over previous
import functools

import jax
import jax.numpy as jnp
from jax import lax
from jax.experimental import pallas as pl
from jax.experimental.pallas import tpu as pltpu
from jax.experimental.pallas import tpu_sc as plsc

N = 10000
E = 320000
D = 128

NC = 2
NS = 16
NW = NC * NS
CHUNK = 128
EPW = E // NW
NF = EPW // CHUNK
TAIL = EPW - NF * CHUNK
NP = 10240
RPT = NP // NS
GBUF = 2
IBUF = 4

_mesh = plsc.VectorSubcoreMesh(
    core_axis_name="c", subcore_axis_name="s", num_cores=NC, num_subcores=NS
)


def _agg_body(with_cnt, *refs):
    if with_cnt:
        (x_hbm, src_hbm, dst_hbm, zacc_hbm,
         acc_out, cnt_out,
         acc_sh, cnt_sh, sib, dib, rows, sit, dit, rows_t, ones_v, cv,
         sem_si, sem_di, sem_g, sem_s, sem_c, sem_t) = refs
    else:
        (x_hbm, src_hbm, dst_hbm, zacc_hbm,
         acc_out,
         acc_sh, sib, dib, rows, sit, dit, rows_t,
         sem_si, sem_di, sem_g, sem_s, sem_t) = refs

    cid = lax.axis_index("c")
    sid = lax.axis_index("s")
    wid = cid * NS + sid

    r0 = pl.multiple_of(sid * RPT, 8)
    pltpu.sync_copy(zacc_hbm.at[pl.ds(r0, RPT)], acc_sh.at[pl.ds(r0, RPT)])
    if with_cnt:
        z16 = jnp.zeros((16,), jnp.float32)
        o16 = jnp.ones((16,), jnp.float32)

        def zrow(r, c):
            cv[pl.ds(r * 16, 16)] = z16
            return c

        lax.fori_loop(0, RPT // 16, zrow, 0)

        def orow(r, c):
            ones_v[pl.ds(r * 16, 16)] = o16
            return c

        lax.fori_loop(0, CHUNK // 16, orow, 0)
        pltpu.sync_copy(cv, cnt_sh.at[pl.ds(r0, RPT)])
    plsc.subcore_barrier()

    base0 = wid * EPW

    def i_descs(i, j):
        base = pl.multiple_of(base0 + i * CHUNK, 8)
        return (
            pltpu.make_async_copy(
                src_hbm.at[pl.ds(base, CHUNK)], sib.at[j], sem_si[j]),
            pltpu.make_async_copy(
                dst_hbm.at[pl.ds(base, CHUNK)], dib.at[j], sem_di[j]),
        )

    def i_start(i, j):
        s, d = i_descs(i, j)
        s.start()
        d.start()

    def i_wait(i, j):
        s, d = i_descs(i, j)
        s.wait()
        d.wait()

    def g_desc(j, b):
        return pltpu.make_async_copy(x_hbm.at[sib.at[j]], rows[b], sem_g[b])

    def s_desc(j, b):
        return pltpu.make_async_copy(
            rows[b], acc_sh.at[dib.at[j]], sem_s[b])

    def c_desc(j, b):
        return pltpu.make_async_copy(ones_v, cnt_sh.at[dib.at[j]], sem_c[b])

    def s_start(j, b):
        s_desc(j, b).start(add=True)
        if with_cnt:
            c_desc(j, b).start(add=True)

    def s_wait(j, b):
        s_desc(j, b).wait()
        if with_cnt:
            c_desc(j, b).wait()

    tbase = pl.multiple_of(base0 + NF * CHUNK, 8)
    t_si = pltpu.make_async_copy(
        src_hbm.at[pl.ds(tbase, TAIL)], sit, sem_t[0])
    t_di = pltpu.make_async_copy(
        dst_hbm.at[pl.ds(tbase, TAIL)], dit, sem_t[1])
    t_g = pltpu.make_async_copy(x_hbm.at[sit], rows_t, sem_t[2])
    t_s = pltpu.make_async_copy(rows_t, acc_sh.at[dit], sem_t[3])
    if with_cnt:
        t_c = pltpu.make_async_copy(
            ones_v.at[pl.ds(0, TAIL)], cnt_sh.at[dit], sem_t[4])

    t_si.start()
    t_di.start()
    for i in range(IBUF):
        i_start(i, i)
    t_si.wait()
    t_di.wait()
    t_g.start()
    i_wait(0, 0)
    g_desc(0, 0).start()
    i_wait(1, 1)
    g_desc(0, 0).wait()
    s_start(0, 0)
    g_desc(1, 1).start()
    for i in (1, 2):
        g_desc(i % IBUF, i % GBUF).wait()
        s_start(i % IBUF, i % GBUF)
        s_wait((i - 1) % IBUF, (i - 1) % GBUF)
        i_start(i + 3, (i + 3) % IBUF)
        i_wait(i + 1, (i + 1) % IBUF)
        g_desc((i + 1) % IBUF, (i + 1) % GBUF).start()

    @pl.loop(3, NF - IBUF + 1, step=IBUF)
    def ring(i0):
        for u in range(IBUF):
            i = i0 + u
            ji = (3 + u) % IBUF
            b = (3 + u) % GBUF
            jp = (2 + u) % IBUF
            bp = (2 + u) % GBUF
            jn = u % IBUF
            bn = u % GBUF
            g_desc(ji, b).wait()
            s_start(ji, b)
            s_wait(jp, bp)
            i_start(i + IBUF - 1, jp)
            i_wait(i + 1, jn)
            g_desc(jn, bn).start()

    for i in range(NF - IBUF + 1, NF):
        g_desc(i % IBUF, i % GBUF).wait()
        s_start(i % IBUF, i % GBUF)
        s_wait((i - 1) % IBUF, (i - 1) % GBUF)
        if i + 1 < NF:
            i_wait(i + 1, (i + 1) % IBUF)
            g_desc((i + 1) % IBUF, (i + 1) % GBUF).start()
    t_g.wait()
    t_s.start(add=True)
    if with_cnt:
        t_c.start(add=True)
    s_wait((NF - 1) % IBUF, (NF - 1) % GBUF)
    t_s.wait()
    if with_cnt:
        t_c.wait()
    plsc.subcore_barrier()

    pltpu.sync_copy(acc_sh.at[pl.ds(r0, RPT)], acc_out.at[cid, pl.ds(r0, RPT)])
    if with_cnt:
        pltpu.sync_copy(cnt_sh.at[pl.ds(r0, RPT)], cv)
        pltpu.sync_copy(cv, cnt_out.at[pl.ds(cid * NP + r0, RPT)])


_agg_with_cnt = pl.kernel(
    functools.partial(_agg_body, True),
    out_type=(
        jax.ShapeDtypeStruct((NC, NP, D), jnp.float32),
        jax.ShapeDtypeStruct((NC * NP,), jnp.float32),
    ),
    mesh=_mesh,
    scratch_types=[
        pltpu.VMEM_SHARED((NP, D), jnp.float32),
        pltpu.VMEM_SHARED((NP,), jnp.float32),
        pltpu.VMEM((IBUF, CHUNK), jnp.int32),
        pltpu.VMEM((IBUF, CHUNK), jnp.int32),
        [pltpu.VMEM((CHUNK, D), jnp.float32) for _ in range(GBUF)],
        pltpu.VMEM((TAIL,), jnp.int32),
        pltpu.VMEM((TAIL,), jnp.int32),
        pltpu.VMEM((TAIL, D), jnp.float32),
        pltpu.VMEM((CHUNK,), jnp.float32),
        pltpu.VMEM((RPT,), jnp.float32),
        [pltpu.SemaphoreType.DMA for _ in range(IBUF)],
        [pltpu.SemaphoreType.DMA for _ in range(IBUF)],
        [pltpu.SemaphoreType.DMA for _ in range(GBUF)],
        [pltpu.SemaphoreType.DMA for _ in range(GBUF)],
        [pltpu.SemaphoreType.DMA for _ in range(GBUF)],
        [pltpu.SemaphoreType.DMA for _ in range(5)],
    ],
    name="sage_agg_cnt",
)

_agg_no_cnt = pl.kernel(
    functools.partial(_agg_body, False),
    out_type=jax.ShapeDtypeStruct((NC, NP, D), jnp.float32),
    mesh=_mesh,
    scratch_types=[
        pltpu.VMEM_SHARED((NP, D), jnp.float32),
        pltpu.VMEM((IBUF, CHUNK), jnp.int32),
        pltpu.VMEM((IBUF, CHUNK), jnp.int32),
        [pltpu.VMEM((CHUNK, D), jnp.float32) for _ in range(GBUF)],
        pltpu.VMEM((TAIL,), jnp.int32),
        pltpu.VMEM((TAIL,), jnp.int32),
        pltpu.VMEM((TAIL, D), jnp.float32),
        [pltpu.SemaphoreType.DMA for _ in range(IBUF)],
        [pltpu.SemaphoreType.DMA for _ in range(IBUF)],
        [pltpu.SemaphoreType.DMA for _ in range(GBUF)],
        [pltpu.SemaphoreType.DMA for _ in range(GBUF)],
        [pltpu.SemaphoreType.DMA for _ in range(4)],
    ],
    name="sage_agg",
)

BN = 400


def _combine_body(p_ref, c_ref, x_ref, wl_ref, wr_ref, b_ref, o_ref):
    cnt = jnp.maximum(c_ref[...], 1.0)
    agg = (p_ref[0] + p_ref[1]) / cnt
    acc = jax.lax.dot_general(
        agg, wl_ref[...], (((1,), (0,)), ((), ())),
        preferred_element_type=jnp.float32)
    acc = acc + jax.lax.dot_general(
        x_ref[...], wr_ref[...], (((1,), (0,)), ((), ())),
        preferred_element_type=jnp.float32)
    o_ref[...] = jnp.maximum(acc + b_ref[...], 0.0)


def _combine(p, c, x, W_l, W_r, b):
    return pl.pallas_call(
        _combine_body,
        grid=(N // BN,),
        in_specs=[
            pl.BlockSpec((NC, BN, D), lambda i: (0, i, 0)),
            pl.BlockSpec((BN, D), lambda i: (i, 0)),
            pl.BlockSpec((BN, D), lambda i: (i, 0)),
            pl.BlockSpec((D, D), lambda i: (0, 0)),
            pl.BlockSpec((D, D), lambda i: (0, 0)),
            pl.BlockSpec((1, D), lambda i: (0, 0)),
        ],
        out_specs=pl.BlockSpec((BN, D), lambda i: (i, 0)),
        out_shape=jax.ShapeDtypeStruct((N, D), jnp.float32),
    )(p, c, x, W_l, W_r, b)


@jax.jit
def kernel(x, edge_index, W1_l, W1_r, b1, W2_l, W2_r, b2):
    src = edge_index[0]
    dst = edge_index[1]
    zacc = jnp.zeros((NP, D), jnp.float32)
    b1r = b1.reshape(1, D)
    b2r = b2.reshape(1, D)

    p1, cnt_flat = _agg_with_cnt(x, src, dst, zacc)
    cnts = cnt_flat.reshape(NC, NP)
    cnt = jnp.broadcast_to((cnts[0] + cnts[1])[:, None], (NP, D))
    h = _combine(p1, cnt, x, W1_l, W1_r, b1r)
    p2 = _agg_no_cnt(h, src, dst, zacc)
    return _combine(p2, cnt, h, W2_l, W2_r, b2r)


# speedup vs baseline: 3.2844x; 1.0024x over previous
import functools

import jax
import jax.numpy as jnp
from jax import lax
from jax.experimental import pallas as pl
from jax.experimental.pallas import tpu as pltpu
from jax.experimental.pallas import tpu_sc as plsc

N = 10000
E = 320000
D = 128

NC = 2
NS = 16
NW = NC * NS
CHUNK = 128
EPW = E // NW
NF = EPW // CHUNK
TAIL = EPW - NF * CHUNK
NP = 10240
RPT = NP // NS
GBUF = 2
IBUF = 4

_mesh = plsc.VectorSubcoreMesh(
    core_axis_name="c", subcore_axis_name="s", num_cores=NC, num_subcores=NS
)


def _agg_body(with_cnt, *refs):
    if with_cnt:
        (x_hbm, src_hbm, dst_hbm, zacc_hbm,
         acc_out,
         acc_sh, cnt_sh, sib, dib, rows, sit, dit, rows_t, ones_v, cv, cvw,
         sem_si, sem_di, sem_g, sem_s, sem_c, sem_t) = refs
    else:
        (x_hbm, src_hbm, dst_hbm, zacc_hbm,
         acc_out,
         acc_sh, sib, dib, rows, sit, dit, rows_t,
         sem_si, sem_di, sem_g, sem_s, sem_t) = refs

    cid = lax.axis_index("c")
    sid = lax.axis_index("s")
    wid = cid * NS + sid

    r0 = pl.multiple_of(sid * RPT, 8)
    pltpu.sync_copy(zacc_hbm.at[pl.ds(r0, RPT)], acc_sh.at[pl.ds(r0, RPT)])
    if with_cnt:
        z16 = jnp.zeros((16,), jnp.float32)
        o16 = jnp.ones((16,), jnp.float32)

        def zrow(r, c):
            cv[pl.ds(r * 16, 16)] = z16
            return c

        lax.fori_loop(0, RPT // 16, zrow, 0)

        def orow(r, c):
            ones_v[pl.ds(r * 16, 16)] = o16
            return c

        lax.fori_loop(0, CHUNK // 16, orow, 0)
        pltpu.sync_copy(cv, cnt_sh.at[pl.ds(r0, RPT)])
    plsc.subcore_barrier()

    base0 = wid * EPW

    def i_descs(i, j):
        base = pl.multiple_of(base0 + i * CHUNK, 8)
        return (
            pltpu.make_async_copy(
                src_hbm.at[pl.ds(base, CHUNK)], sib.at[j], sem_si[j]),
            pltpu.make_async_copy(
                dst_hbm.at[pl.ds(base, CHUNK)], dib.at[j], sem_di[j]),
        )

    def i_start(i, j):
        s, d = i_descs(i, j)
        s.start()
        d.start()

    def i_wait(i, j):
        s, d = i_descs(i, j)
        s.wait()
        d.wait()

    def g_desc(j, b):
        return pltpu.make_async_copy(x_hbm.at[sib.at[j]], rows[b], sem_g[b])

    def s_desc(j, b):
        return pltpu.make_async_copy(
            rows[b], acc_sh.at[dib.at[j]], sem_s[b])

    def c_desc(j, b):
        return pltpu.make_async_copy(ones_v, cnt_sh.at[dib.at[j]], sem_c[b])

    def s_start(j, b):
        s_desc(j, b).start(add=True)
        if with_cnt:
            c_desc(j, b).start(add=True)

    def s_wait(j, b):
        s_desc(j, b).wait()
        if with_cnt:
            c_desc(j, b).wait()

    tbase = pl.multiple_of(base0 + NF * CHUNK, 8)
    t_si = pltpu.make_async_copy(
        src_hbm.at[pl.ds(tbase, TAIL)], sit, sem_t[0])
    t_di = pltpu.make_async_copy(
        dst_hbm.at[pl.ds(tbase, TAIL)], dit, sem_t[1])
    t_g = pltpu.make_async_copy(x_hbm.at[sit], rows_t, sem_t[2])
    t_s = pltpu.make_async_copy(rows_t, acc_sh.at[dit], sem_t[3])
    if with_cnt:
        t_c = pltpu.make_async_copy(
            ones_v.at[pl.ds(0, TAIL)], cnt_sh.at[dit], sem_t[4])

    t_si.start()
    t_di.start()
    for i in range(IBUF):
        i_start(i, i)
    t_si.wait()
    t_di.wait()
    t_g.start()
    i_wait(0, 0)
    g_desc(0, 0).start()
    i_wait(1, 1)
    g_desc(0, 0).wait()
    s_start(0, 0)
    g_desc(1, 1).start()
    for i in (1, 2):
        g_desc(i % IBUF, i % GBUF).wait()
        s_start(i % IBUF, i % GBUF)
        s_wait((i - 1) % IBUF, (i - 1) % GBUF)
        i_start(i + 3, (i + 3) % IBUF)
        i_wait(i + 1, (i + 1) % IBUF)
        g_desc((i + 1) % IBUF, (i + 1) % GBUF).start()

    @pl.loop(3, NF - IBUF + 1, step=IBUF)
    def ring(i0):
        for u in range(IBUF):
            i = i0 + u
            ji = (3 + u) % IBUF
            b = (3 + u) % GBUF
            jp = (2 + u) % IBUF
            bp = (2 + u) % GBUF
            jn = u % IBUF
            bn = u % GBUF
            g_desc(ji, b).wait()
            s_start(ji, b)
            s_wait(jp, bp)
            i_start(i + IBUF - 1, jp)
            i_wait(i + 1, jn)
            g_desc(jn, bn).start()

    for i in range(NF - IBUF + 1, NF):
        g_desc(i % IBUF, i % GBUF).wait()
        s_start(i % IBUF, i % GBUF)
        s_wait((i - 1) % IBUF, (i - 1) % GBUF)
        if i + 1 < NF:
            i_wait(i + 1, (i + 1) % IBUF)
            g_desc((i + 1) % IBUF, (i + 1) % GBUF).start()
    t_g.wait()
    t_s.start(add=True)
    if with_cnt:
        t_c.start(add=True)
    s_wait((NF - 1) % IBUF, (NF - 1) % GBUF)
    t_s.wait()
    if with_cnt:
        t_c.wait()
    plsc.subcore_barrier()

    pltpu.sync_copy(acc_sh.at[pl.ds(r0, RPT)], acc_out.at[cid, pl.ds(r0, RPT)])
    if with_cnt:
        plsc.subcore_barrier()
        pltpu.sync_copy(cnt_sh.at[pl.ds(r0, RPT)], cv)
        for r in range(5):
            for c in range(8):
                cvw[r, pl.ds(c * 16, 16)] = cv[pl.ds((r * 8 + c) * 16, 16)]
        pltpu.sync_copy(
            cvw, acc_out.at[cid, pl.ds(pl.multiple_of(N + sid * 8, 8), 8)])


_agg_with_cnt = pl.kernel(
    functools.partial(_agg_body, True),
    out_type=jax.ShapeDtypeStruct((NC, NP, D), jnp.float32),
    mesh=_mesh,
    scratch_types=[
        pltpu.VMEM_SHARED((NP, D), jnp.float32),
        pltpu.VMEM_SHARED((NP,), jnp.float32),
        pltpu.VMEM((IBUF, CHUNK), jnp.int32),
        pltpu.VMEM((IBUF, CHUNK), jnp.int32),
        [pltpu.VMEM((CHUNK, D), jnp.float32) for _ in range(GBUF)],
        pltpu.VMEM((TAIL,), jnp.int32),
        pltpu.VMEM((TAIL,), jnp.int32),
        pltpu.VMEM((TAIL, D), jnp.float32),
        pltpu.VMEM((CHUNK,), jnp.float32),
        pltpu.VMEM((RPT,), jnp.float32),
        pltpu.VMEM((8, D), jnp.float32),
        [pltpu.SemaphoreType.DMA for _ in range(IBUF)],
        [pltpu.SemaphoreType.DMA for _ in range(IBUF)],
        [pltpu.SemaphoreType.DMA for _ in range(GBUF)],
        [pltpu.SemaphoreType.DMA for _ in range(GBUF)],
        [pltpu.SemaphoreType.DMA for _ in range(GBUF)],
        [pltpu.SemaphoreType.DMA for _ in range(5)],
    ],
    name="sage_agg_cnt",
)

_agg_no_cnt = pl.kernel(
    functools.partial(_agg_body, False),
    out_type=jax.ShapeDtypeStruct((NC, NP, D), jnp.float32),
    mesh=_mesh,
    scratch_types=[
        pltpu.VMEM_SHARED((NP, D), jnp.float32),
        pltpu.VMEM((IBUF, CHUNK), jnp.int32),
        pltpu.VMEM((IBUF, CHUNK), jnp.int32),
        [pltpu.VMEM((CHUNK, D), jnp.float32) for _ in range(GBUF)],
        pltpu.VMEM((TAIL,), jnp.int32),
        pltpu.VMEM((TAIL,), jnp.int32),
        pltpu.VMEM((TAIL, D), jnp.float32),
        [pltpu.SemaphoreType.DMA for _ in range(IBUF)],
        [pltpu.SemaphoreType.DMA for _ in range(IBUF)],
        [pltpu.SemaphoreType.DMA for _ in range(GBUF)],
        [pltpu.SemaphoreType.DMA for _ in range(GBUF)],
        [pltpu.SemaphoreType.DMA for _ in range(4)],
    ],
    name="sage_agg",
)

BN = 400


def _combine_body(p_ref, c_ref, x_ref, wl_ref, wr_ref, b_ref, o_ref):
    cnt = jnp.maximum(c_ref[...], 1.0)
    agg = (p_ref[0] + p_ref[1]) / cnt
    acc = jax.lax.dot_general(
        agg, wl_ref[...], (((1,), (0,)), ((), ())),
        preferred_element_type=jnp.float32)
    acc = acc + jax.lax.dot_general(
        x_ref[...], wr_ref[...], (((1,), (0,)), ((), ())),
        preferred_element_type=jnp.float32)
    o_ref[...] = jnp.maximum(acc + b_ref[...], 0.0)


def _combine(p, c, x, W_l, W_r, b):
    return pl.pallas_call(
        _combine_body,
        grid=(N // BN,),
        in_specs=[
            pl.BlockSpec((NC, BN, D), lambda i: (0, i, 0)),
            pl.BlockSpec((BN, D), lambda i: (i, 0)),
            pl.BlockSpec((BN, D), lambda i: (i, 0)),
            pl.BlockSpec((D, D), lambda i: (0, 0)),
            pl.BlockSpec((D, D), lambda i: (0, 0)),
            pl.BlockSpec((1, D), lambda i: (0, 0)),
        ],
        out_specs=pl.BlockSpec((BN, D), lambda i: (i, 0)),
        out_shape=jax.ShapeDtypeStruct((N, D), jnp.float32),
    )(p, c, x, W_l, W_r, b)


@jax.jit
def kernel(x, edge_index, W1_l, W1_r, b1, W2_l, W2_r, b2):
    src = edge_index[0]
    dst = edge_index[1]
    zacc = jnp.zeros((NP, D), jnp.float32)
    b1r = b1.reshape(1, D)
    b2r = b2.reshape(1, D)

    p1 = _agg_with_cnt(x, src, dst, zacc)
    cnts = p1[:, N:N + NS * 8].reshape(NC, NS, 8, D)[:, :, :5].reshape(NC, NP)
    cnt = jnp.broadcast_to((cnts[0] + cnts[1])[:, None], (NP, D))
    h = _combine(p1, cnt, x, W1_l, W1_r, b1r)
    p2 = _agg_no_cnt(h, src, dst, zacc)
    return _combine(p2, cnt, h, W2_l, W2_r, b2r)
